# Initial kernel scaffold; baseline (speedup 1.0000x reference)
import jax
import jax.numpy as jnp
from jax.experimental import pallas as pl


def kernel(inputs):
    raise NotImplementedError("write your pallas kernel here")


# speedup vs baseline: 1.1760x; 1.1760x over previous
import jax
import jax.numpy as jnp
from jax.experimental import pallas as pl

_REPEAT = 16


def _unpool_body(x_ref, o_ref):
    x = x_ref[...]
    bb, ub = x.shape
    y = jax.lax.broadcast_in_dim(x, (bb, ub, _REPEAT), (0, 1))
    o_ref[...] = y.reshape(bb, ub * _REPEAT)


def kernel(inputs):
    b, u = inputs.shape
    bb, ub = 8, 2048
    grid = (b // bb, u // ub)
    return pl.pallas_call(
        _unpool_body,
        grid=grid,
        in_specs=[pl.BlockSpec((bb, ub), lambda i, j: (i, j))],
        out_specs=pl.BlockSpec((bb, ub * _REPEAT), lambda i, j: (i, j)),
        out_shape=jax.ShapeDtypeStruct((b, u * _REPEAT), inputs.dtype),
    )(inputs)


# speedup vs baseline: 3.0446x; 2.5890x over previous
import numpy as np
import jax
import jax.numpy as jnp
from jax.experimental import pallas as pl

_REPEAT = 16
_K = 128


def _expand_matrix():
    e = np.zeros((_K, _K * _REPEAT), dtype=np.float32)
    for q in range(_K):
        e[q, q * _REPEAT:(q + 1) * _REPEAT] = 1.0
    return jnp.asarray(e)


def _unpool_body(x_ref, e_ref, o_ref):
    x = x_ref[0]
    o_ref[0] = jnp.dot(x, e_ref[...], preferred_element_type=jnp.float32)


def kernel(inputs):
    b, u = inputs.shape
    p = u // _K
    x3 = inputs.reshape(b, p, _K)
    e = _expand_matrix()
    out = pl.pallas_call(
        _unpool_body,
        grid=(b,),
        in_specs=[
            pl.BlockSpec((1, p, _K), lambda i: (i, 0, 0)),
            pl.BlockSpec((_K, _K * _REPEAT), lambda i: (0, 0)),
        ],
        out_specs=pl.BlockSpec((1, p, _K * _REPEAT), lambda i: (i, 0, 0)),
        out_shape=jax.ShapeDtypeStruct((b, p, _K * _REPEAT), inputs.dtype),
    )(x3, e)
    return out.reshape(b, u * _REPEAT)


# speedup vs baseline: 9.1924x; 3.0192x over previous
import numpy as np
import jax
import jax.numpy as jnp
from jax.experimental import pallas as pl

_REPEAT = 16
_K = 128


def _expand_matrix():
    e = np.zeros((_K, _K * _REPEAT), dtype=np.float32)
    for q in range(_K):
        e[q, q * _REPEAT:(q + 1) * _REPEAT] = 1.0
    return jnp.asarray(e)


def _unpool_body(x_ref, e_ref, o_ref):
    o_ref[...] = jnp.dot(x_ref[...], e_ref[...],
                         preferred_element_type=jnp.float32)


def kernel(inputs):
    b, u = inputs.shape
    n_blocks = u // _K
    return pl.pallas_call(
        _unpool_body,
        grid=(n_blocks,),
        in_specs=[
            pl.BlockSpec((b, _K), lambda j: (0, j)),
            pl.BlockSpec((_K, _K * _REPEAT), lambda j: (0, 0)),
        ],
        out_specs=pl.BlockSpec((b, _K * _REPEAT), lambda j: (0, j)),
        out_shape=jax.ShapeDtypeStruct((b, u * _REPEAT), inputs.dtype),
    )(inputs, _expand_matrix())


# speedup vs baseline: 12.4170x; 1.3508x over previous
import numpy as np
import jax
import jax.numpy as jnp
from jax.experimental import pallas as pl

_REPEAT = 16
_K = 128


def _expand_matrix():
    e = np.zeros((_K, _K * _REPEAT), dtype=np.float32)
    for q in range(_K):
        e[q, q * _REPEAT:(q + 1) * _REPEAT] = 1.0
    return jnp.asarray(e)


_KCHUNKS = 2


def _unpool_body(x_ref, e_ref, o_ref):
    e = e_ref[...]
    for c in range(_KCHUNKS):
        o_ref[:, c * _K * _REPEAT:(c + 1) * _K * _REPEAT] = jnp.dot(
            x_ref[:, c * _K:(c + 1) * _K], e,
            preferred_element_type=jnp.float32)


def kernel(inputs):
    b, u = inputs.shape
    n_blocks = u // (_K * _KCHUNKS)
    return pl.pallas_call(
        _unpool_body,
        grid=(n_blocks,),
        in_specs=[
            pl.BlockSpec((b, _K * _KCHUNKS), lambda j: (0, j)),
            pl.BlockSpec((_K, _K * _REPEAT), lambda j: (0, 0)),
        ],
        out_specs=pl.BlockSpec((b, _K * _REPEAT * _KCHUNKS),
                               lambda j: (0, j)),
        out_shape=jax.ShapeDtypeStruct((b, u * _REPEAT), inputs.dtype),
    )(inputs, _expand_matrix())


# speedup vs baseline: 14.5890x; 1.1749x over previous
import numpy as np
import jax
import jax.numpy as jnp
from jax.experimental import pallas as pl

_REPEAT = 16
_K = 128


def _expand_matrix():
    e = np.zeros((_K, _K * _REPEAT), dtype=np.float32)
    for q in range(_K):
        e[q, q * _REPEAT:(q + 1) * _REPEAT] = 1.0
    return jnp.asarray(e)


_KCHUNKS = 4


def _unpool_body(x_ref, e_ref, o_ref):
    e = e_ref[...]
    for c in range(_KCHUNKS):
        o_ref[:, c * _K * _REPEAT:(c + 1) * _K * _REPEAT] = jnp.dot(
            x_ref[:, c * _K:(c + 1) * _K], e,
            preferred_element_type=jnp.float32)


def kernel(inputs):
    b, u = inputs.shape
    n_blocks = u // (_K * _KCHUNKS)
    return pl.pallas_call(
        _unpool_body,
        grid=(n_blocks,),
        in_specs=[
            pl.BlockSpec((b, _K * _KCHUNKS), lambda j: (0, j)),
            pl.BlockSpec((_K, _K * _REPEAT), lambda j: (0, 0)),
        ],
        out_specs=pl.BlockSpec((b, _K * _REPEAT * _KCHUNKS),
                               lambda j: (0, j)),
        out_shape=jax.ShapeDtypeStruct((b, u * _REPEAT), inputs.dtype),
    )(inputs, _expand_matrix())


# speedup vs baseline: 14.6885x; 1.0068x over previous
import numpy as np
import jax
import jax.numpy as jnp
from jax.experimental import pallas as pl

_REPEAT = 16
_K = 128


def _expand_matrix():
    e = np.zeros((_K, _K * _REPEAT), dtype=np.float32)
    for q in range(_K):
        e[q, q * _REPEAT:(q + 1) * _REPEAT] = 1.0
    return jnp.asarray(e)


_KCHUNKS = 8


def _unpool_body(x_ref, e_ref, o_ref):
    e = e_ref[...]
    for c in range(_KCHUNKS):
        o_ref[:, c * _K * _REPEAT:(c + 1) * _K * _REPEAT] = jnp.dot(
            x_ref[:, c * _K:(c + 1) * _K], e,
            preferred_element_type=jnp.float32)


def kernel(inputs):
    b, u = inputs.shape
    n_blocks = u // (_K * _KCHUNKS)
    return pl.pallas_call(
        _unpool_body,
        grid=(n_blocks,),
        in_specs=[
            pl.BlockSpec((b, _K * _KCHUNKS), lambda j: (0, j)),
            pl.BlockSpec((_K, _K * _REPEAT), lambda j: (0, 0)),
        ],
        out_specs=pl.BlockSpec((b, _K * _REPEAT * _KCHUNKS),
                               lambda j: (0, j)),
        out_shape=jax.ShapeDtypeStruct((b, u * _REPEAT), inputs.dtype),
    )(inputs, _expand_matrix())
